# Initial kernel scaffold; baseline (speedup 1.0000x reference)
import jax
import jax.numpy as jnp
from jax.experimental import pallas as pl


def kernel(x, target_vec, table, W, b):
    raise NotImplementedError("write your pallas kernel here")


# speedup vs baseline: 2.4167x; 2.4167x over previous
import functools
import math

import jax
import jax.numpy as jnp
from jax import lax
from jax.experimental import pallas as pl
from jax.experimental.pallas import tpu as pltpu
from jax.experimental.pallas import tpu_sc as plsc

D_MODEL = 128
SCALE = math.sqrt(float(D_MODEL))

_NC = 2
_NS = 16
_NW = _NC * _NS
_L = 16

CHUNK = 128


def _make_gather(N, D):
    assert N % (_NW * CHUNK) == 0
    n_per_w = N // _NW
    n_chunks = n_per_w // CHUNK

    mesh = plsc.VectorSubcoreMesh(core_axis_name="c", subcore_axis_name="s")

    @functools.partial(
        pl.kernel,
        mesh=mesh,
        out_type=jax.ShapeDtypeStruct((N, D), jnp.float32),
        scratch_types=[
            pltpu.VMEM((n_chunks, CHUNK), jnp.int32),
            pltpu.VMEM((CHUNK, D), jnp.float32),
            pltpu.SemaphoreType.DMA,
        ],
    )
    def gather_kernel(table_hbm, idx_hbm, out_hbm, idx_v, buf, sem):
        wid = lax.axis_index("s") * _NC + lax.axis_index("c")
        base = wid * n_per_w
        pltpu.sync_copy(idx_hbm.at[wid], idx_v)

        def chunk_body(c, carry):
            pltpu.async_copy(table_hbm.at[idx_v.at[c]], buf, sem).wait()

            def row_body(r, carry2):
                for j in range(D // _L):
                    sl = pl.ds(j * _L, _L)
                    buf[r, sl] = buf[r, sl] * SCALE
                return carry2

            lax.fori_loop(0, CHUNK, row_body, 0)
            pltpu.sync_copy(buf, out_hbm.at[pl.ds(base + c * CHUNK, CHUNK)])
            return carry

        lax.fori_loop(0, n_chunks, chunk_body, 0)

    return gather_kernel


def kernel(x, target_vec, table, W, b):
    B, S = x.shape
    V, D = table.shape
    N = B * S
    n_per_w = N // _NW
    n_chunks = n_per_w // CHUNK
    idx = x.reshape(_NW, n_chunks, CHUNK).astype(jnp.int32)
    out = _make_gather(N, D)(table, idx)
    return out.reshape(B, S, D)
